# baseline (speedup 1.0000x reference)
import functools

import jax
import jax.numpy as jnp
from jax import lax
from jax.experimental import pallas as pl
from jax.experimental.pallas import tpu as pltpu
from jax.experimental.pallas import tpu_sc as plsc

VOCAB = 100000
D_MODEL = 64

_NC = 2
_NS = 16
_NW = _NC * _NS
_L = 16

_NSEQ = 4096
_SEQLEN = 200
_S_PER_W = _NSEQ // _NW
_PCH = 2
_N_CH = _SEQLEN // _PCH
_ROWS = _PCH * _S_PER_W


def _emb_body(seq_hbm, table_hbm, out_hbm,
              idx_v, idxt_v, rows_v, tbuf_v, gsem, osem):
    wid = lax.axis_index("s") * _NC + lax.axis_index("c")
    s0 = wid * _S_PER_W

    pltpu.sync_copy(seq_hbm.at[pl.ds(s0, _S_PER_W)], idx_v)

    lanes = lax.iota(jnp.int32, _L)

    def build_idxt(p0, b):
        for pi in range(_PCH):
            col = jnp.full((_L,), p0 + pi, jnp.int32)
            for sb in range(_S_PER_W // _L):
                rows = lanes + (sb * _L)
                v = plsc.load_gather(idx_v, [rows, col])
                idxt_v[b, pl.ds(pi * _S_PER_W + sb * _L, _L)] = v

    def start_gather(p0, b):
        build_idxt(p0, b)
        pltpu.async_copy(table_hbm.at[idxt_v.at[b]], rows_v.at[b],
                         gsem.at[b])

    def wait_gather(b):
        pltpu.make_async_copy(table_hbm.at[idxt_v.at[b]], rows_v.at[b],
                              gsem.at[b]).wait()

    def transpose(b):
        for pi in range(_PCH):
            def dbody(d, carry):
                dcol = jnp.full((_L,), d, jnp.int32)
                for sb in range(_S_PER_W // _L):
                    r = lanes + (pi * _S_PER_W + sb * _L)
                    v = plsc.load_gather(rows_v.at[b], [r, dcol])
                    tbuf_v[b, pi, d, pl.ds(sb * _L, _L)] = v
                return carry
            lax.fori_loop(0, D_MODEL, dbody, 0)

    def start_store(p0, b):
        pltpu.async_copy(
            tbuf_v.at[b],
            out_hbm.at[pl.ds(p0, _PCH), :, pl.ds(s0, _S_PER_W)],
            osem.at[b])

    def wait_store(p0, b):
        pltpu.make_async_copy(
            tbuf_v.at[b],
            out_hbm.at[pl.ds(p0, _PCH), :, pl.ds(s0, _S_PER_W)],
            osem.at[b]).wait()

    start_gather(0, 0)
    start_gather(_PCH, 1)
    for b in range(2):
        p0 = b * _PCH
        wait_gather(b)
        transpose(b)
        start_store(p0, b)
        start_gather(p0 + 2 * _PCH, b)

    def outer(o, carry):
        for b in range(2):
            p0 = (2 * o + b) * _PCH
            wait_gather(b)
            wait_store(p0 - 2 * _PCH, b)
            transpose(b)
            start_store(p0, b)
            start_gather(p0 + 2 * _PCH, b)
        return carry

    lax.fori_loop(1, _N_CH // 2 - 1, outer, 0)

    for b in range(2):
        p0 = (_N_CH - 2 + b) * _PCH
        wait_gather(b)
        wait_store(p0 - 2 * _PCH, b)
        transpose(b)
        start_store(p0, b)
        wait_store(p0, b)


_emb = functools.partial(
    pl.kernel,
    out_type=jax.ShapeDtypeStruct((_SEQLEN, D_MODEL, _NSEQ), jnp.float32),
    mesh=plsc.VectorSubcoreMesh(core_axis_name="c", subcore_axis_name="s"),
    scratch_types=[
        pltpu.VMEM((_S_PER_W, _SEQLEN), jnp.int32),
        pltpu.VMEM((2, _ROWS), jnp.int32),
        pltpu.VMEM((2, _ROWS, D_MODEL), jnp.float32),
        pltpu.VMEM((2, _PCH, D_MODEL, _S_PER_W), jnp.float32),
        pltpu.SemaphoreType.DMA((2,)),
        pltpu.SemaphoreType.DMA((2,)),
    ],
    compiler_params=pltpu.CompilerParams(use_tc_tiling_on_sc=False,
                                         needs_layout_passes=False),
)(_emb_body)


def kernel(sequence, lookup):
    out_t = _emb(sequence.astype(jnp.int32), lookup)
    return jnp.transpose(out_t, (2, 0, 1))
